# Initial kernel scaffold; baseline (speedup 1.0000x reference)
import functools

import jax
import jax.numpy as jnp
from jax import lax
from jax.experimental import pallas as pl
from jax.experimental.pallas import tpu as pltpu
from jax.experimental.pallas import tpu_sc as plsc

N_NODES = 10000
N_EDGES = 320000
HIDDEN = 16

NC = 2
NS = 16
NW = NC * NS
CHUNK = 128
N_CHUNKS = N_EDGES // CHUNK
ROWS_PER_S = N_NODES // NS

_MESH = plsc.VectorSubcoreMesh(core_axis_name="c", subcore_axis_name="s")


def _make_sc_spmm(width):
    with_deg = width == 32

    @functools.partial(
        pl.kernel,
        out_type=jax.ShapeDtypeStruct((NC, N_NODES, width), jnp.float32),
        mesh=_MESH,
        scratch_types=[
            pltpu.VMEM((CHUNK,), jnp.int32),
            pltpu.VMEM((CHUNK,), jnp.int32),
            pltpu.VMEM((CHUNK,), jnp.float32),
            pltpu.VMEM((CHUNK, HIDDEN), jnp.float32),
            pltpu.VMEM((CHUNK, width), jnp.float32),
            pltpu.VMEM_SHARED((N_NODES, width), jnp.float32),
            pltpu.SemaphoreType.DMA,
        ],
    )
    def sc_spmm(h_hbm, src_hbm, dst_hbm, a_hbm, zeros_hbm, out_hbm,
                src_v, dst_v, a_v, rows_v, msg_v, acc_sh, sem):
        cid = lax.axis_index("c")
        sid = lax.axis_index("s")
        wid = sid * NC + cid

        row0 = sid * ROWS_PER_S
        pltpu.sync_copy(zeros_hbm.at[pl.ds(row0, ROWS_PER_S)],
                        acc_sh.at[pl.ds(row0, ROWS_PER_S)])
        plsc.subcore_barrier()

        @pl.loop(wid, N_CHUNKS, step=NW)
        def _chunk(c):
            base = c * CHUNK
            pltpu.sync_copy(src_hbm.at[pl.ds(base, CHUNK)], src_v)
            pltpu.sync_copy(dst_hbm.at[pl.ds(base, CHUNK)], dst_v)
            pltpu.sync_copy(a_hbm.at[pl.ds(base, CHUNK)], a_v)
            pltpu.async_copy(h_hbm.at[src_v], rows_v, sem).wait()

            @pl.loop(0, CHUNK)
            def _edge(e):
                a16 = plsc.load_gather(a_v, [jnp.full((16,), e, jnp.int32)])
                msg_v[e, 0:16] = rows_v[e, 0:16] * a16
                if with_deg:
                    msg_v[e, 16:32] = a16

            pltpu.sync_copy(msg_v, acc_sh.at[dst_v], add=True)

        plsc.subcore_barrier()
        pltpu.sync_copy(acc_sh.at[pl.ds(row0, ROWS_PER_S)],
                        out_hbm.at[cid, pl.ds(row0, ROWS_PER_S)])

    return sc_spmm


_sc_spmm32 = _make_sc_spmm(32)
_sc_spmm16 = _make_sc_spmm(16)


def _tc_in_proj(x, w1t, b1):
    def body(x_ref, w_ref, b_ref, o_ref):
        o_ref[...] = jnp.dot(x_ref[...], w_ref[...],
                             preferred_element_type=jnp.float32) + b_ref[...]
    return pl.pallas_call(
        body,
        out_shape=jax.ShapeDtypeStruct((N_NODES, HIDDEN), jnp.float32),
    )(x, w1t, b1)


def _tc_relu_sum(o1):
    def body(o1_ref, r_ref):
        s = o1_ref[0, :, 0:HIDDEN] + o1_ref[1, :, 0:HIDDEN]
        r_ref[...] = jnp.maximum(s, 0.0)
    return pl.pallas_call(
        body,
        out_shape=jax.ShapeDtypeStruct((N_NODES, HIDDEN), jnp.float32),
    )(o1)


def _tc_out_proj(o2, o1, w2t, b2):
    def body(o2_ref, o1_ref, w_ref, b_ref, out_ref):
        s2 = o2_ref[0] + o2_ref[1]
        deg = o1_ref[0, :, HIDDEN:HIDDEN + 1] + o1_ref[1, :, HIDDEN:HIDDEN + 1]
        out_ref[...] = (jnp.dot(s2, w_ref[...],
                                preferred_element_type=jnp.float32)
                        + deg * b_ref[...])
    return pl.pallas_call(
        body,
        out_shape=jax.ShapeDtypeStruct((N_NODES, 128), jnp.float32),
    )(o2, o1, w2t, b2)


def kernel(X, edge_index, A_vals, W1_w, W1_b, W2_w, W2_b):
    dst = edge_index[0].astype(jnp.int32)
    src = edge_index[1].astype(jnp.int32)
    w1t = W1_w.T
    b1 = W1_b[None, :]
    w2t = W2_w.T
    b2 = W2_b[None, :]
    zeros32 = jnp.zeros((N_NODES, 32), jnp.float32)
    zeros16 = jnp.zeros((N_NODES, HIDDEN), jnp.float32)

    h = _tc_in_proj(X, w1t, b1)
    o1 = _sc_spmm32(h, src, dst, A_vals, zeros32)
    r = _tc_relu_sum(o1)
    o2 = _sc_spmm16(r, src, dst, A_vals, zeros16)
    return _tc_out_proj(o2, o1, w2t, b2)


# speedup vs baseline: 6.1925x; 6.1925x over previous
import dataclasses
import functools

import jax
import jax.numpy as jnp
from jax import lax
from jax.experimental import pallas as pl
from jax.experimental.pallas import tpu as pltpu
from jax.experimental.pallas import tpu_sc as plsc

N_NODES = 10000
N_EDGES = 320000
HIDDEN = 16

NC = 2
NS = 16
NW = NC * NS
CHUNK = 128
N_CHUNKS = N_EDGES // CHUNK
ROWS_PER_S = 624
TAIL_ROW0 = ROWS_PER_S * NS
TAIL_ROWS = N_NODES - TAIL_ROW0

_MESH = plsc.VectorSubcoreMesh(core_axis_name="c", subcore_axis_name="s")

_SC_PARAMS = pltpu.CompilerParams(use_tc_tiling_on_sc=False)
if "needs_layout_passes" in pltpu.CompilerParams.__dataclass_fields__:
    _SC_PARAMS = dataclasses.replace(_SC_PARAMS, needs_layout_passes=False)


def _make_sc_spmm(width):
    with_deg = width == 32

    @functools.partial(
        pl.kernel,
        out_type=jax.ShapeDtypeStruct((NC, N_NODES, width), jnp.float32),
        mesh=_MESH,
        scratch_types=[
            pltpu.VMEM((CHUNK,), jnp.int32),
            pltpu.VMEM((CHUNK,), jnp.int32),
            pltpu.VMEM((CHUNK,), jnp.float32),
            pltpu.VMEM((CHUNK, HIDDEN), jnp.float32),
            pltpu.VMEM((CHUNK, width), jnp.float32),
            pltpu.VMEM_SHARED((N_NODES, width), jnp.float32),
            pltpu.SemaphoreType.DMA,
        ],
        compiler_params=_SC_PARAMS,
    )
    def sc_spmm(h_hbm, src_hbm, dst_hbm, a_hbm, zeros_hbm, out_hbm,
                src_v, dst_v, a_v, rows_v, msg_v, acc_sh, sem):
        cid = lax.axis_index("c")
        sid = lax.axis_index("s")
        wid = sid * NC + cid

        row0 = sid * ROWS_PER_S
        pltpu.sync_copy(zeros_hbm.at[pl.ds(row0, ROWS_PER_S)],
                        acc_sh.at[pl.ds(row0, ROWS_PER_S)])

        @pl.when(sid == NS - 1)
        def _zero_tail():
            pltpu.sync_copy(zeros_hbm.at[pl.ds(TAIL_ROW0, TAIL_ROWS)],
                            acc_sh.at[pl.ds(TAIL_ROW0, TAIL_ROWS)])

        plsc.subcore_barrier()

        @pl.loop(wid, N_CHUNKS, step=NW)
        def _chunk(c):
            base = c * CHUNK
            pltpu.sync_copy(src_hbm.at[pl.ds(base, CHUNK)], src_v)
            pltpu.sync_copy(dst_hbm.at[pl.ds(base, CHUNK)], dst_v)
            pltpu.sync_copy(a_hbm.at[pl.ds(base, CHUNK)], a_v)
            pltpu.async_copy(h_hbm.at[src_v], rows_v, sem).wait()

            @pl.loop(0, CHUNK)
            def _edge(e):
                a16 = plsc.load_gather(a_v, [jnp.full((16,), e, jnp.int32)])
                msg_v[e, 0:16] = rows_v[e, 0:16] * a16
                if with_deg:
                    msg_v[e, 16:32] = a16

            pltpu.sync_copy(msg_v, acc_sh.at[dst_v], add=True)

        plsc.subcore_barrier()
        pltpu.sync_copy(acc_sh.at[pl.ds(row0, ROWS_PER_S)],
                        out_hbm.at[cid, pl.ds(row0, ROWS_PER_S)])

        @pl.when(sid == NS - 1)
        def _drain_tail():
            pltpu.sync_copy(acc_sh.at[pl.ds(TAIL_ROW0, TAIL_ROWS)],
                            out_hbm.at[cid, pl.ds(TAIL_ROW0, TAIL_ROWS)])

    return sc_spmm


_sc_spmm32 = _make_sc_spmm(32)
_sc_spmm16 = _make_sc_spmm(16)


def _tc_in_proj(x, w1t, b1):
    def body(x_ref, w_ref, b_ref, o_ref):
        o_ref[...] = jnp.dot(x_ref[...], w_ref[...],
                             preferred_element_type=jnp.float32) + b_ref[...]
    return pl.pallas_call(
        body,
        out_shape=jax.ShapeDtypeStruct((N_NODES, HIDDEN), jnp.float32),
    )(x, w1t, b1)


def _tc_relu_sum(o1):
    def body(o1_ref, r_ref):
        s = o1_ref[0, :, 0:HIDDEN] + o1_ref[1, :, 0:HIDDEN]
        r_ref[...] = jnp.maximum(s, 0.0)
    return pl.pallas_call(
        body,
        out_shape=jax.ShapeDtypeStruct((N_NODES, HIDDEN), jnp.float32),
    )(o1)


def _tc_out_proj(o2, o1, w2t, b2):
    def body(o2_ref, o1_ref, w_ref, b_ref, out_ref):
        s2 = o2_ref[0] + o2_ref[1]
        deg = o1_ref[0, :, HIDDEN:HIDDEN + 1] + o1_ref[1, :, HIDDEN:HIDDEN + 1]
        out_ref[...] = (jnp.dot(s2, w_ref[...],
                                preferred_element_type=jnp.float32)
                        + deg * b_ref[...])
    return pl.pallas_call(
        body,
        out_shape=jax.ShapeDtypeStruct((N_NODES, 128), jnp.float32),
    )(o2, o1, w2t, b2)


def kernel(X, edge_index, A_vals, W1_w, W1_b, W2_w, W2_b):
    dst = edge_index[0].astype(jnp.int32)
    src = edge_index[1].astype(jnp.int32)
    w1t = W1_w.T
    b1 = W1_b[None, :]
    w2t = W2_w.T
    b2 = W2_b[None, :]
    zeros32 = jnp.zeros((N_NODES, 32), jnp.float32)
    zeros16 = jnp.zeros((N_NODES, HIDDEN), jnp.float32)

    h = _tc_in_proj(X, w1t, b1)
    o1 = _sc_spmm32(h, src, dst, A_vals, zeros32)
    r = _tc_relu_sum(o1)
    o2 = _sc_spmm16(r, src, dst, A_vals, zeros16)
    return _tc_out_proj(o2, o1, w2t, b2)


# speedup vs baseline: 14.7365x; 2.3797x over previous
import dataclasses
import functools

import jax
import jax.numpy as jnp
from jax import lax
from jax.experimental import pallas as pl
from jax.experimental.pallas import tpu as pltpu
from jax.experimental.pallas import tpu_sc as plsc

N_NODES = 10000
N_EDGES = 320000
HIDDEN = 16

NC = 2
NS = 16
NW = NC * NS
CHUNK = 128
CPW = 80
PAD_CHUNKS = NW * CPW
PAD_E = PAD_CHUNKS * CHUNK
ROWS_PER_S = 624
TAIL_ROW0 = ROWS_PER_S * NS
TAIL_ROWS = N_NODES - TAIL_ROW0

_MESH = plsc.VectorSubcoreMesh(core_axis_name="c", subcore_axis_name="s")

_SC_PARAMS = pltpu.CompilerParams(use_tc_tiling_on_sc=False)
if "needs_layout_passes" in pltpu.CompilerParams.__dataclass_fields__:
    _SC_PARAMS = dataclasses.replace(_SC_PARAMS, needs_layout_passes=False)


def _make_sc_spmm(width):
    with_deg = width == 32

    @functools.partial(
        pl.kernel,
        out_type=jax.ShapeDtypeStruct((NC, N_NODES, width), jnp.float32),
        mesh=_MESH,
        scratch_types=[
            pltpu.VMEM((CPW, CHUNK), jnp.int32),
            pltpu.VMEM((CPW, CHUNK), jnp.int32),
            pltpu.VMEM((CPW * CHUNK,), jnp.float32),
            pltpu.VMEM((CHUNK, HIDDEN), jnp.float32),
            pltpu.VMEM((CHUNK, HIDDEN), jnp.float32),
            pltpu.VMEM((CHUNK, width), jnp.float32),
            pltpu.VMEM((CHUNK, width), jnp.float32),
            pltpu.VMEM_SHARED((N_NODES, width), jnp.float32),
            pltpu.SemaphoreType.DMA,
            pltpu.SemaphoreType.DMA,
            pltpu.SemaphoreType.DMA,
            pltpu.SemaphoreType.DMA,
        ],
        compiler_params=_SC_PARAMS,
    )
    def sc_spmm(h_hbm, src_hbm, dst_hbm, a_hbm, zeros_hbm, out_hbm,
                srcb, dstb, av, rows0, rows1, msg0, msg1, acc_sh,
                sg0, sg1, ss0, ss1):
        rows = (rows0, rows1)
        msg = (msg0, msg1)
        sem_g = (sg0, sg1)
        sem_s = (ss0, ss1)
        cid = lax.axis_index("c")
        sid = lax.axis_index("s")
        wid = sid * NC + cid
        c0 = wid * CPW

        pltpu.sync_copy(src_hbm.at[pl.ds(c0, CPW)], srcb)
        pltpu.sync_copy(dst_hbm.at[pl.ds(c0, CPW)], dstb)
        pltpu.sync_copy(a_hbm.at[pl.ds(c0 * CHUNK, CPW * CHUNK)], av)

        row0 = sid * ROWS_PER_S
        pltpu.sync_copy(zeros_hbm.at[pl.ds(row0, ROWS_PER_S)],
                        acc_sh.at[pl.ds(row0, ROWS_PER_S)])

        @pl.when(sid == NS - 1)
        def _zero_tail():
            pltpu.sync_copy(zeros_hbm.at[pl.ds(TAIL_ROW0, TAIL_ROWS)],
                            acc_sh.at[pl.ds(TAIL_ROW0, TAIL_ROWS)])

        plsc.subcore_barrier()

        pltpu.async_copy(h_hbm.at[srcb.at[0]], rows[0], sem_g[0])

        @pl.loop(0, CPW // 2)
        def _blk(k):
            for p in (0, 1):
                c = 2 * k + p

                @pl.when(c + 1 < CPW)
                def _fire_next_gather():
                    pltpu.async_copy(h_hbm.at[srcb.at[c + 1]],
                                     rows[1 - p], sem_g[1 - p])

                pltpu.make_async_copy(h_hbm.at[pl.ds(0, CHUNK)],
                                      rows[p], sem_g[p]).wait()

                @pl.when(c >= 2)
                def _drain_scatter():
                    pltpu.make_async_copy(zeros_hbm.at[pl.ds(0, CHUNK)],
                                          msg[p], sem_s[p]).wait()

                abase = c * CHUNK

                @pl.loop(0, CHUNK, unroll=8)
                def _edge(e):
                    a16 = plsc.load_gather(
                        av, [jnp.full((16,), abase + e, jnp.int32)])
                    msg[p][e, 0:16] = rows[p][e, 0:16] * a16
                    if with_deg:
                        msg[p][e, 16:32] = a16

                pltpu.async_copy(msg[p], acc_sh.at[dstb.at[c]],
                                 sem_s[p], add=True)

        for p in (0, 1):
            pltpu.make_async_copy(zeros_hbm.at[pl.ds(0, CHUNK)],
                                  msg[p], sem_s[p]).wait()

        plsc.subcore_barrier()
        pltpu.sync_copy(acc_sh.at[pl.ds(row0, ROWS_PER_S)],
                        out_hbm.at[cid, pl.ds(row0, ROWS_PER_S)])

        @pl.when(sid == NS - 1)
        def _drain_tail():
            pltpu.sync_copy(acc_sh.at[pl.ds(TAIL_ROW0, TAIL_ROWS)],
                            out_hbm.at[cid, pl.ds(TAIL_ROW0, TAIL_ROWS)])

    return sc_spmm


_sc_spmm32 = _make_sc_spmm(32)
_sc_spmm16 = _make_sc_spmm(16)


def _tc_in_proj(x, w1t, b1):
    def body(x_ref, w_ref, b_ref, o_ref):
        o_ref[...] = jnp.dot(x_ref[...], w_ref[...],
                             preferred_element_type=jnp.float32) + b_ref[...]
    return pl.pallas_call(
        body,
        out_shape=jax.ShapeDtypeStruct((N_NODES, HIDDEN), jnp.float32),
    )(x, w1t, b1)


def _tc_relu_sum(o1):
    def body(o1_ref, r_ref):
        s = o1_ref[0, :, 0:HIDDEN] + o1_ref[1, :, 0:HIDDEN]
        r_ref[...] = jnp.maximum(s, 0.0)
    return pl.pallas_call(
        body,
        out_shape=jax.ShapeDtypeStruct((N_NODES, HIDDEN), jnp.float32),
    )(o1)


def _tc_out_proj(o2, o1, w2t, b2):
    def body(o2_ref, o1_ref, w_ref, b_ref, out_ref):
        s2 = o2_ref[0] + o2_ref[1]
        deg = o1_ref[0, :, HIDDEN:HIDDEN + 1] + o1_ref[1, :, HIDDEN:HIDDEN + 1]
        out_ref[...] = (jnp.dot(s2, w_ref[...],
                                preferred_element_type=jnp.float32)
                        + deg * b_ref[...])
    return pl.pallas_call(
        body,
        out_shape=jax.ShapeDtypeStruct((N_NODES, 128), jnp.float32),
    )(o2, o1, w2t, b2)


def kernel(X, edge_index, A_vals, W1_w, W1_b, W2_w, W2_b):
    pad = PAD_E - N_EDGES
    dst = jnp.pad(edge_index[0].astype(jnp.int32), (0, pad)).reshape(
        PAD_CHUNKS, CHUNK)
    src = jnp.pad(edge_index[1].astype(jnp.int32), (0, pad)).reshape(
        PAD_CHUNKS, CHUNK)
    A_vals = jnp.pad(A_vals, (0, pad))
    w1t = W1_w.T
    b1 = W1_b[None, :]
    w2t = W2_w.T
    b2 = W2_b[None, :]
    zeros32 = jnp.zeros((N_NODES, 32), jnp.float32)
    zeros16 = jnp.zeros((N_NODES, HIDDEN), jnp.float32)

    h = _tc_in_proj(X, w1t, b1)
    o1 = _sc_spmm32(h, src, dst, A_vals, zeros32)
    r = _tc_relu_sum(o1)
    o2 = _sc_spmm16(r, src, dst, A_vals, zeros16)
    return _tc_out_proj(o2, o1, w2t, b2)
